# baseline (speedup 1.0000x reference)
import functools

import jax
import jax.numpy as jnp
from jax import lax
from jax.experimental import pallas as pl
from jax.experimental.pallas import tpu as pltpu
from jax.experimental.pallas import tpu_sc as plsc

NC, NS = 2, 16
NW = NC * NS
LANES = 16
PB = 128


@functools.cache
def _tpe_kernel(B, L, D):
    NPB = L // PB
    NBH = NW // NPB
    BH = B // NBH
    assert NPB * PB == L and NBH * BH == B and D % LANES == 0
    mesh = plsc.VectorSubcoreMesh(core_axis_name="c", subcore_axis_name="s")
    NCH = PB // LANES

    @functools.partial(
        pl.kernel,
        out_type=jax.ShapeDtypeStruct((B * L, D), jnp.float32),
        mesh=mesh,
        scratch_types=[
            pltpu.VMEM((BH, PB), jnp.int32),
            pltpu.VMEM((PB, D), jnp.float32),
            pltpu.VMEM((PB, D), jnp.float32),
        ] + [pltpu.SemaphoreType.DMA] * (PB // LANES),
        compiler_params=pltpu.CompilerParams(needs_layout_passes=False),
    )
    def k(x_hbm, tok_hbm, pos_hbm, out_hbm, idx_v, pos_v, out_v, *gsems):
        wid = lax.axis_index("s") * NC + lax.axis_index("c")
        pb = lax.rem(wid, NPB)
        bh = wid // NPB
        pbase = pb * PB
        b0 = bh * BH
        pltpu.sync_copy(pos_hbm.at[pl.ds(pbase, PB)], pos_v)
        pltpu.sync_copy(x_hbm.at[pl.ds(b0, BH), pl.ds(pbase, PB)], idx_v)

        def fire_chunk(b, i):
            tvec = idx_v[b, pl.ds(i * LANES, LANES)]
            for kk in range(LANES):
                t = tvec[kk]
                pltpu.async_copy(
                    tok_hbm.at[t // 8, lax.rem(t, 8)],
                    out_v.at[i * LANES + kk], gsems[i % 8])

        def drain_chunk(i):
            pltpu.make_async_copy(
                out_hbm.at[pl.ds(0, LANES)],
                out_v.at[pl.ds(i * LANES, LANES), :], gsems[i % 8]).wait()

        def add_pos_chunk(i):
            for kk in range(LANES):
                r = i * LANES + kk
                for j in range(D // LANES):
                    sl = pl.ds(j * LANES, LANES)
                    plsc.addupdate(out_v.at[r, sl], pos_v[r, sl])

        AHEAD = 7

        def per_batch(b, carry):
            for i in range(AHEAD):
                fire_chunk(b, i)
            for i in range(NCH):
                if i + AHEAD < NCH:
                    fire_chunk(b, i + AHEAD)
                drain_chunk(i)
                add_pos_chunk(i)
            pltpu.sync_copy(out_v, out_hbm.at[pl.ds((b0 + b) * L + pbase, PB)])
            return carry

        lax.fori_loop(0, BH, per_batch, 0)

    return k


def kernel(x, token_table, pos_table):
    B, L = x.shape
    V, D = token_table.shape
    tok3 = token_table.reshape(V // 8, 8, D)
    flat = _tpe_kernel(B, L, D)(x.astype(jnp.int32), tok3, pos_table)
    return flat.reshape(B, L, D)


# speedup vs baseline: 1.0072x; 1.0072x over previous
import functools

import jax
import jax.numpy as jnp
from jax import lax
from jax.experimental import pallas as pl
from jax.experimental.pallas import tpu as pltpu
from jax.experimental.pallas import tpu_sc as plsc

NC, NS = 2, 16
NW = NC * NS
LANES = 16
PB = 128


@functools.cache
def _tpe_kernel(B, L, D):
    NPB = L // PB
    NBH = NW // NPB
    BH = B // NBH
    assert NPB * PB == L and NBH * BH == B and D % LANES == 0
    mesh = plsc.VectorSubcoreMesh(core_axis_name="c", subcore_axis_name="s")
    NCH = PB // LANES

    @functools.partial(
        pl.kernel,
        out_type=jax.ShapeDtypeStruct((B * L, D), jnp.float32),
        mesh=mesh,
        scratch_types=[
            pltpu.VMEM((BH, PB), jnp.int32),
            pltpu.VMEM((8 * LANES, D), jnp.float32),
            pltpu.VMEM((PB, D), jnp.float32),
            pltpu.VMEM((2, PB, D), jnp.float32),
        ] + [pltpu.SemaphoreType.DMA] * (PB // LANES),
        compiler_params=pltpu.CompilerParams(needs_layout_passes=False),
    )
    def k(x_hbm, tok_hbm, pos_hbm, out_hbm, idx_v, blk_v, pos_v, out_v,
          *gsems):
        wid = lax.axis_index("s") * NC + lax.axis_index("c")
        pb = lax.rem(wid, NPB)
        bh = wid // NPB
        pbase = pb * PB
        b0 = bh * BH
        pltpu.sync_copy(pos_hbm.at[pl.ds(pbase, PB)], pos_v)
        pltpu.sync_copy(x_hbm.at[pl.ds(b0, BH), pl.ds(pbase, PB)], idx_v)

        NPAIR = 2 * NCH

        def fire_chunk(b0p, c):
            b = b0p + c // NCH
            i = c % NCH
            slot = (c % 8) * LANES
            tvec = idx_v[b, pl.ds(i * LANES, LANES)]
            for kk in range(LANES):
                t = tvec[kk]
                pltpu.async_copy(
                    tok_hbm.at[t // 8, lax.rem(t, 8)],
                    blk_v.at[slot + kk], gsems[c % 8])

        def drain_chunk(c):
            slot = (c % 8) * LANES
            pltpu.make_async_copy(
                out_hbm.at[pl.ds(0, LANES)],
                blk_v.at[pl.ds(slot, LANES), :], gsems[c % 8]).wait()

        def extract_chunk(c):
            par = c // NCH
            i = c % NCH
            slot = (c % 8) * LANES
            for kk in range(LANES):
                r = i * LANES + kk
                for j in range(D // LANES):
                    sl = pl.ds(j * LANES, LANES)
                    out_v[par, r, sl] = blk_v[slot + kk, sl] + pos_v[r, sl]

        AHEAD = 7

        def per_pair(p, carry):
            b0p = 2 * p
            for c in range(AHEAD):
                fire_chunk(b0p, c)
            for c in range(NPAIR):
                if c + AHEAD < NPAIR:
                    fire_chunk(b0p, c + AHEAD)
                drain_chunk(c)
                extract_chunk(c)
                if c == NCH - 1:
                    pltpu.sync_copy(
                        out_v.at[0],
                        out_hbm.at[pl.ds((b0 + b0p) * L + pbase, PB)])
            pltpu.sync_copy(
                out_v.at[1],
                out_hbm.at[pl.ds((b0 + b0p + 1) * L + pbase, PB)])
            return carry

        lax.fori_loop(0, BH // 2, per_pair, 0)

    return k


def kernel(x, token_table, pos_table):
    B, L = x.shape
    V, D = token_table.shape
    tok3 = token_table.reshape(V // 8, 8, D)
    flat = _tpe_kernel(B, L, D)(x.astype(jnp.int32), tok3, pos_table)
    return flat.reshape(B, L, D)


# speedup vs baseline: 1.0591x; 1.0515x over previous
import functools

import jax
import jax.numpy as jnp
from jax import lax
from jax.experimental import pallas as pl
from jax.experimental.pallas import tpu as pltpu
from jax.experimental.pallas import tpu_sc as plsc

NC, NS = 2, 16
NW = NC * NS
LANES = 16
PB = 128


@functools.cache
def _tpe_kernel(B, L, D):
    NPB = L // PB
    NBH = NW // NPB
    BH = B // NBH
    assert NPB * PB == L and NBH * BH == B and D % LANES == 0
    mesh = plsc.VectorSubcoreMesh(core_axis_name="c", subcore_axis_name="s")
    NCH = PB // LANES

    @functools.partial(
        pl.kernel,
        out_type=jax.ShapeDtypeStruct((B * L, D), jnp.float32),
        mesh=mesh,
        scratch_types=[
            pltpu.VMEM((BH, PB), jnp.int32),
            pltpu.VMEM((8 * LANES, D), jnp.float32),
            pltpu.VMEM((PB, D), jnp.float32),
            pltpu.VMEM((PB, D), jnp.float32),
        ] + [pltpu.SemaphoreType.DMA] * (PB // LANES),
        compiler_params=pltpu.CompilerParams(needs_layout_passes=False),
    )
    def k(x_hbm, tok_hbm, pos_hbm, out_hbm, idx_v, blk_v, pos_v, out_v,
          *gsems):
        wid = lax.axis_index("s") * NC + lax.axis_index("c")
        pb = lax.rem(wid, NPB)
        bh = wid // NPB
        pbase = pb * PB
        b0 = bh * BH
        pltpu.sync_copy(pos_hbm.at[pl.ds(pbase, PB)], pos_v)
        pltpu.sync_copy(x_hbm.at[pl.ds(b0, BH), pl.ds(pbase, PB)], idx_v)

        def fire_chunk(b, i):
            slot = (i % 8) * LANES
            tvec = idx_v[b, pl.ds(i * LANES, LANES)]
            for kk in range(LANES):
                t = tvec[kk]
                pltpu.async_copy(
                    tok_hbm.at[t // 8, lax.rem(t, 8)],
                    blk_v.at[slot + kk], gsems[i % 8])

        def drain_chunk(i):
            slot = (i % 8) * LANES
            pltpu.make_async_copy(
                out_hbm.at[pl.ds(0, LANES)],
                blk_v.at[pl.ds(slot, LANES), :], gsems[i % 8]).wait()

        def extract_chunk(b, i):
            slot = (i % 8) * LANES
            for kk in range(LANES):
                r = i * LANES + kk
                for j in range(D // LANES):
                    sl = pl.ds(j * LANES, LANES)
                    out_v[r, sl] = blk_v[slot + kk, sl] + pos_v[r, sl]

        AHEAD = 6

        def per_batch(b, carry):
            for i in range(AHEAD):
                fire_chunk(b, i)
            for i in range(NCH):
                if i + AHEAD < NCH:
                    fire_chunk(b, i + AHEAD)
                drain_chunk(i)
                extract_chunk(b, i)
            pltpu.sync_copy(out_v, out_hbm.at[pl.ds((b0 + b) * L + pbase, PB)])
            return carry

        lax.fori_loop(0, BH, per_batch, 0)

    return k


def kernel(x, token_table, pos_table):
    B, L = x.shape
    V, D = token_table.shape
    tok3 = token_table.reshape(V // 8, 8, D)
    flat = _tpe_kernel(B, L, D)(x.astype(jnp.int32), tok3, pos_table)
    return flat.reshape(B, L, D)
